# baseline (speedup 1.0000x reference)
import functools

import jax
import jax.numpy as jnp
from jax import lax
from jax.experimental import pallas as pl
from jax.experimental.pallas import tpu as pltpu
from jax.experimental.pallas import tpu_sc as plsc

H = 64
B = 16384
NC = 2
NS = 16
NW = NC * NS
BPW = B // NW
L = 16


def _gate_sc(x, wbt, theta_flat):
    mesh = plsc.VectorSubcoreMesh(core_axis_name="c", subcore_axis_name="s")

    @functools.partial(
        pl.kernel,
        out_type=jax.ShapeDtypeStruct((B,), jnp.float32),
        mesh=mesh,
        scratch_types=[
            pltpu.VMEM((BPW,), jnp.int32),
            pltpu.VMEM((BPW * H,), jnp.float32),
            pltpu.VMEM((H + 1, L), jnp.float32),
            pltpu.VMEM((BPW,), jnp.float32),
            pltpu.SemaphoreType.DMA,
        ],
        compiler_params=pltpu.CompilerParams(
            needs_layout_passes=False, disable_bounds_checks=True),
    )
    def k(x_hbm, wbt_hbm, theta_hbm, out_hbm, idx_v, rows_v, wbt_v, out_v,
          sem):
        wid = lax.axis_index("s") * NC + lax.axis_index("c")
        base = wid * BPW
        pltpu.sync_copy(x_hbm.at[pl.ds(base, BPW)], idx_v)
        pltpu.sync_copy(wbt_hbm, wbt_v)

        def fire_body(g, carry):
            off = pl.multiple_of(g * L, L)
            v = idx_v[pl.ds(off, L)]
            for l in range(L):
                pltpu.async_copy(
                    theta_hbm.at[pl.ds(pl.multiple_of(v[l] * H, H), H)],
                    rows_v.at[pl.ds((off + l) * H, H)],
                    sem,
                )
            return carry

        lax.fori_loop(0, BPW // L, fire_body, 0)
        pltpu.make_async_copy(
            theta_hbm.at[pl.ds(0, BPW * H)], rows_v, sem).wait()

        lanes = lax.iota(jnp.int32, L)
        bv = wbt_v[H, :]
        zero = jnp.zeros((L,), jnp.float32)

        def dot_body(g, carry):
            row0 = pl.multiple_of(g * L, L)
            fbase = (row0 + lanes) * H
            acc = [bv, zero, zero, zero]
            for j in range(H):
                col = plsc.load_gather(rows_v, [fbase + j])
                acc[j % 4] = acc[j % 4] + col * wbt_v[j, :]
            s = (acc[0] + acc[1]) + (acc[2] + acc[3])
            out_v[pl.ds(row0, L)] = 1.0 / (1.0 + jnp.exp(-s))
            return carry

        lax.fori_loop(0, BPW // L, dot_body, 0)
        pltpu.sync_copy(out_v, out_hbm.at[pl.ds(base, BPW)])

    return k(x, wbt, theta_flat)


def kernel(X, Y, gate_theta, W, b):
    wbt = jnp.concatenate(
        [jnp.broadcast_to(W[:, 0][:, None], (H, L)),
         jnp.broadcast_to(b, (1, L))]).astype(jnp.float32)
    out = _gate_sc(X, wbt, gate_theta.reshape(-1))
    return out[:, None]


# speedup vs baseline: 3.4306x; 3.4306x over previous
import functools

import jax
import jax.numpy as jnp
from jax import lax
from jax.experimental import pallas as pl
from jax.experimental.pallas import tpu as pltpu
from jax.experimental.pallas import tpu_sc as plsc

NUME = 1000000
H = 64
B = 16384
NC = 2
NS = 16
NW = NC * NS
BPW = B // NW
CHUNK = 128
NCHUNK = BPW // CHUNK
CB = 8192
NBLK = (NUME + CB - 1) // CB


def _matvec_tc(theta_t, wrow, b2d):
    def body(w_ref, b_ref, blk_ref, o_ref):
        s = jnp.dot(w_ref[...], blk_ref[...],
                    preferred_element_type=jnp.float32)
        o_ref[...] = jax.nn.sigmoid(s + b_ref[0, 0])

    return pl.pallas_call(
        body,
        grid=(NBLK,),
        in_specs=[
            pl.BlockSpec((1, H), lambda i: (0, 0)),
            pl.BlockSpec(memory_space=pltpu.SMEM),
            pl.BlockSpec((H, CB), lambda i: (0, i)),
        ],
        out_specs=pl.BlockSpec((1, CB), lambda i: (0, i)),
        out_shape=jax.ShapeDtypeStruct((1, NUME), jnp.float32),
    )(wrow, b2d, theta_t)


def _gather_sc(x, gates):
    mesh = plsc.VectorSubcoreMesh(core_axis_name="c", subcore_axis_name="s")

    @functools.partial(
        pl.kernel,
        out_type=jax.ShapeDtypeStruct((B,), jnp.float32),
        mesh=mesh,
        scratch_types=[
            pltpu.VMEM((BPW,), jnp.int32),
            pltpu.VMEM((BPW,), jnp.float32),
            pltpu.SemaphoreType.DMA,
        ],
        compiler_params=pltpu.CompilerParams(
            needs_layout_passes=False, disable_bounds_checks=True),
    )
    def k(x_hbm, gates_hbm, out_hbm, idx_v, out_v, sem):
        wid = lax.axis_index("s") * NC + lax.axis_index("c")
        base = wid * BPW
        pltpu.sync_copy(x_hbm.at[pl.ds(base, BPW)], idx_v)
        copies = [
            pltpu.async_copy(
                gates_hbm.at[idx_v.at[pl.ds(c * CHUNK, CHUNK)]],
                out_v.at[pl.ds(c * CHUNK, CHUNK)],
                sem,
            )
            for c in range(NCHUNK)
        ]
        for cp in copies:
            cp.wait()
        pltpu.sync_copy(out_v, out_hbm.at[pl.ds(base, BPW)])

    return k(x, gates)


def kernel(X, Y, gate_theta, W, b):
    theta_t = gate_theta.T
    wrow = W.reshape(1, H)
    b2d = b.reshape(1, 1)
    gates = _matvec_tc(theta_t, wrow, b2d)
    out = _gather_sc(X, gates.reshape(NUME))
    return out[:, None]
